# baseline (speedup 1.0000x reference)
import functools

import jax
import jax.numpy as jnp
from jax.experimental import pallas as pl
from jax.experimental.pallas import tpu as pltpu
from jax.experimental.pallas import tpu_sc as plsc

_B = 1024
_F = 32
_C = 100000
_INV_T = 20.0
_CHUNK = 10000
_NCHUNK = _C // _CHUNK
_SUB = 20
_SUBROWS = _CHUNK // _SUB

_PACK = 128 // _F
_NW = 16
_B_PER_W = _B // _NW
_L = 16


def _sc_gather_kernel(idx_hbm, table_hbm, out_hbm, idx_v, idx4_v, rows_v,
                      sem):
    wid = jax.lax.axis_index("s") + jax.lax.axis_index("c") * 0
    base = wid * _B_PER_W
    pltpu.sync_copy(idx_hbm.at[pl.ds(base, _B_PER_W)], idx_v)
    for h in range(_B_PER_W // _L):
        sl = pl.ds(h * _L, _L)
        idx4_v[sl] = jax.lax.shift_right_logical(idx_v[sl], 2)
    pltpu.async_copy(table_hbm.at[idx4_v], rows_v, sem).wait()
    pltpu.sync_copy(rows_v, out_hbm.at[pl.ds(base, _B_PER_W)])


def _gather_rows(targets, bank128):
    mesh = plsc.VectorSubcoreMesh(core_axis_name="c", subcore_axis_name="s", num_cores=1)
    run = functools.partial(
        pl.kernel,
        mesh=mesh,
        out_type=jax.ShapeDtypeStruct((_B, _PACK * _F), jnp.float32),
        scratch_types=[
            pltpu.VMEM((_B_PER_W,), jnp.int32),
            pltpu.VMEM((_B_PER_W,), jnp.int32),
            pltpu.VMEM((_B_PER_W, _PACK * _F), jnp.float32),
            pltpu.SemaphoreType.DMA,
        ],
    )(_sc_gather_kernel)
    return run(targets, bank128)


_LOG2E = 1.4426950408889634


def _ce_kernel(inputs_ref, targets_ref, gathered_ref, bank_ref, diff_ref,
               xn_ref, m_ref, s_ref):
    c = pl.program_id(0)

    @pl.when(c == 0)
    def _init():
        x = inputs_ref[...]
        n2 = jnp.sum(x * x, axis=1, keepdims=True)
        xn_ref[...] = x * (_INV_T * _LOG2E / jnp.maximum(jnp.sqrt(n2), 1e-12))
        m_ref[...] = jnp.full((1, _B), -1e30, jnp.float32)
        s_ref[...] = jnp.zeros((1, _B), jnp.float32)

    xn = xn_ref[...]
    m = m_ref[...]
    s = s_ref[...]
    for t in range(_SUB):
        logits = jax.lax.dot_general(
            bank_ref[t * _SUBROWS:(t + 1) * _SUBROWS, :], xn,
            (((1,), (1,)), ((), ())),
            preferred_element_type=jnp.float32)
        m_new = jnp.maximum(m, jnp.max(logits, axis=0, keepdims=True))
        s = (s * jnp.exp2(m - m_new)
             + jnp.sum(jnp.exp2(logits - m_new), axis=0, keepdims=True))
        m = m_new
    m_ref[...] = m
    s_ref[...] = s

    @pl.when(c == _NCHUNK - 1)
    def _fin():
        lse = 0.6931471805599453 * m_ref[...] + jnp.log(s_ref[...])
        tmod = targets_ref[...] & (_PACK - 1)
        g4 = gathered_ref[...]
        p = jnp.zeros((_B, 1), jnp.float32)
        for k in range(_PACK):
            dk = jnp.sum(xn * g4[:, k * _F:(k + 1) * _F], axis=1,
                         keepdims=True)
            p = jnp.where(tmod == k, dk, p)
        diff_ref[...] = lse - jnp.transpose(p) * (1.0 / _LOG2E)


def kernel(backbone_inputs, inputs, targets, features_bank):
    del backbone_inputs
    tgt = targets.astype(jnp.int32)
    bank128 = features_bank.reshape(_C // _PACK, _PACK * _F)
    gathered = _gather_rows(tgt, bank128)
    diff = pl.pallas_call(
        _ce_kernel,
        grid=(_NCHUNK,),
        in_specs=[
            pl.BlockSpec((_B, _F), lambda c: (0, 0)),
            pl.BlockSpec((_B, 1), lambda c: (0, 0)),
            pl.BlockSpec((_B, _PACK * _F), lambda c: (0, 0)),
            pl.BlockSpec((_CHUNK, _F), lambda c: (c, 0)),
        ],
        out_specs=pl.BlockSpec((1, _B), lambda c: (0, 0)),
        out_shape=jax.ShapeDtypeStruct((1, _B), jnp.float32),
        scratch_shapes=[
            pltpu.VMEM((_B, _F), jnp.float32),
            pltpu.VMEM((1, _B), jnp.float32),
            pltpu.VMEM((1, _B), jnp.float32),
        ],
    )(inputs, tgt.reshape(_B, 1), gathered, features_bank)
    return jnp.mean(diff)


# speedup vs baseline: 1.0123x; 1.0123x over previous
import functools

import jax
import jax.numpy as jnp
from jax.experimental import pallas as pl
from jax.experimental.pallas import tpu as pltpu
from jax.experimental.pallas import tpu_sc as plsc

_B = 1024
_F = 32
_C = 100000
_INV_T = 20.0
_CHUNK = 20000
_NCHUNK = _C // _CHUNK
_SUB = 32
_SUBROWS = _CHUNK // _SUB

_PACK = 128 // _F
_NW = 16
_B_PER_W = _B // _NW
_L = 16


def _sc_gather_kernel(idx_hbm, table_hbm, out_hbm, idx_v, idx4_v, rows_v,
                      sem):
    wid = jax.lax.axis_index("s") + jax.lax.axis_index("c") * 0
    base = wid * _B_PER_W
    pltpu.sync_copy(idx_hbm.at[pl.ds(base, _B_PER_W)], idx_v)
    for h in range(_B_PER_W // _L):
        sl = pl.ds(h * _L, _L)
        idx4_v[sl] = jax.lax.shift_right_logical(idx_v[sl], 2)
    pltpu.async_copy(table_hbm.at[idx4_v], rows_v, sem).wait()
    pltpu.sync_copy(rows_v, out_hbm.at[pl.ds(base, _B_PER_W)])


def _gather_rows(targets, bank128):
    mesh = plsc.VectorSubcoreMesh(core_axis_name="c", subcore_axis_name="s", num_cores=1)
    run = functools.partial(
        pl.kernel,
        mesh=mesh,
        out_type=jax.ShapeDtypeStruct((_B, _PACK * _F), jnp.float32),
        scratch_types=[
            pltpu.VMEM((_B_PER_W,), jnp.int32),
            pltpu.VMEM((_B_PER_W,), jnp.int32),
            pltpu.VMEM((_B_PER_W, _PACK * _F), jnp.float32),
            pltpu.SemaphoreType.DMA,
        ],
    )(_sc_gather_kernel)
    return run(targets, bank128)


_LOG2E = 1.4426950408889634


def _ce_kernel(inputs_ref, targets_ref, gathered_ref, bank_ref, diff_ref,
               xn_ref, m_ref, s_ref):
    c = pl.program_id(0)

    @pl.when(c == 0)
    def _init():
        x = inputs_ref[...]
        n2 = jnp.sum(x * x, axis=1, keepdims=True)
        xn_ref[...] = x * (_INV_T * _LOG2E / jnp.maximum(jnp.sqrt(n2), 1e-12))
        m_ref[...] = jnp.full((1, _B), -1e30, jnp.float32)
        s_ref[...] = jnp.zeros((1, _B), jnp.float32)

    xn = xn_ref[...]
    m = m_ref[...]
    s = s_ref[...]
    for t in range(_SUB):
        logits = jax.lax.dot_general(
            bank_ref[t * _SUBROWS:(t + 1) * _SUBROWS, :], xn,
            (((1,), (1,)), ((), ())),
            preferred_element_type=jnp.float32)
        m_new = jnp.maximum(m, jnp.max(logits, axis=0, keepdims=True))
        s = (s * jnp.exp2(m - m_new)
             + jnp.sum(jnp.exp2(logits - m_new), axis=0, keepdims=True))
        m = m_new
    m_ref[...] = m
    s_ref[...] = s

    @pl.when(c == _NCHUNK - 1)
    def _fin():
        lse = 0.6931471805599453 * m_ref[...] + jnp.log(s_ref[...])
        tmod = targets_ref[...] & (_PACK - 1)
        g4 = gathered_ref[...]
        p = jnp.zeros((_B, 1), jnp.float32)
        for k in range(_PACK):
            dk = jnp.sum(xn * g4[:, k * _F:(k + 1) * _F], axis=1,
                         keepdims=True)
            p = jnp.where(tmod == k, dk, p)
        diff_ref[...] = lse - jnp.transpose(p) * (1.0 / _LOG2E)


def kernel(backbone_inputs, inputs, targets, features_bank):
    del backbone_inputs
    tgt = targets.astype(jnp.int32)
    bank128 = features_bank.reshape(_C // _PACK, _PACK * _F)
    gathered = _gather_rows(tgt, bank128)
    diff = pl.pallas_call(
        _ce_kernel,
        grid=(_NCHUNK,),
        in_specs=[
            pl.BlockSpec((_B, _F), lambda c: (0, 0)),
            pl.BlockSpec((_B, 1), lambda c: (0, 0)),
            pl.BlockSpec((_B, _PACK * _F), lambda c: (0, 0)),
            pl.BlockSpec((_CHUNK, _F), lambda c: (c, 0)),
        ],
        out_specs=pl.BlockSpec((1, _B), lambda c: (0, 0)),
        out_shape=jax.ShapeDtypeStruct((1, _B), jnp.float32),
        scratch_shapes=[
            pltpu.VMEM((_B, _F), jnp.float32),
            pltpu.VMEM((1, _B), jnp.float32),
            pltpu.VMEM((1, _B), jnp.float32),
        ],
    )(inputs, tgt.reshape(_B, 1), gathered, features_bank)
    return jnp.mean(diff)
